# baseline (speedup 1.0000x reference)
import functools

import jax
import jax.numpy as jnp
from jax import lax
from jax.experimental import pallas as pl
from jax.experimental.pallas import tpu as pltpu
from jax.experimental.pallas import tpu_sc as plsc

_NC = 2
_NS = 16
_K = 64
_NB = 3


def _ceil_to(a, m):
    return (a + m - 1) // m * m


def _gcd(a, b):
    while b:
        a, b = b, a % b
    return a


def _sc_degree(n_pad, e_pad, d):
    nw = _NC * _NS
    per_w = e_pad // nw
    iters = per_w // _K
    nblocks = iters // _NB
    rows_per_tile = n_pad // _NS
    mesh = plsc.VectorSubcoreMesh(core_axis_name="c", subcore_axis_name="s")

    scratch = (
        [pltpu.VMEM((_K, d), jnp.float32)] * 2
        + [pltpu.VMEM((_K,), jnp.int32)] * _NB
        + [pltpu.VMEM_SHARED((n_pad, d), jnp.float32)]
        + [pltpu.SemaphoreType.DMA] * _NB
    )

    @functools.partial(
        pl.kernel,
        out_type=jax.ShapeDtypeStruct((_NC, n_pad, d), jnp.float32),
        mesh=mesh,
        scratch_types=scratch,
    )
    def k(ones_hbm, dst_hbm, zrows_hbm, out_hbm, ones_v, zb, *rest):
        dvecs = rest[:_NB]
        deg_sh = rest[_NB]
        sem_i = rest[_NB + 1:2 * _NB + 1]
        c = lax.axis_index("c")
        s = lax.axis_index("s")
        w = c * _NS + s
        row0 = s * rows_per_tile
        pltpu.sync_copy(ones_hbm, ones_v)
        pltpu.sync_copy(zrows_hbm, zb)
        for t in range(rows_per_tile // _K):
            pltpu.sync_copy(zb, deg_sh.at[pl.ds(row0 + t * _K, _K), :])
        zrem = rows_per_tile % _K
        if zrem:
            pltpu.sync_copy(
                zb.at[pl.ds(0, zrem), :],
                deg_sh.at[pl.ds(row0 + rows_per_tile - zrem, zrem), :])
        for b_ in range(_NB):
            pltpu.async_copy(dst_hbm.at[w * iters + b_], dvecs[b_], sem_i[b_])
        plsc.subcore_barrier()

        def body(j, carry):
            for b_ in range(_NB):
                i_ = j * _NB + b_
                pltpu.make_async_copy(
                    dst_hbm.at[w * iters + i_], dvecs[b_], sem_i[b_]).wait()
                pltpu.sync_copy(ones_v, deg_sh.at[dvecs[b_]], add=True)

                @pl.when(j + 1 < nblocks)
                def _():
                    pltpu.async_copy(
                        dst_hbm.at[w * iters + i_ + _NB], dvecs[b_], sem_i[b_])
            return carry
        lax.fori_loop(0, nblocks, body, 0)
        plsc.subcore_barrier()
        pltpu.sync_copy(
            deg_sh.at[pl.ds(row0, rows_per_tile), :],
            out_hbm.at[c, pl.ds(row0, rows_per_tile), :],
        )

    return k


def _sc_aggregate(n_pad, e_pad, d):
    nw = _NC * _NS
    per_w = e_pad // nw
    iters = per_w // _K
    nblocks = iters // _NB
    rows_per_tile = n_pad // _NS
    mesh = plsc.VectorSubcoreMesh(core_axis_name="c", subcore_axis_name="s")

    scratch = (
        [pltpu.VMEM((iters, _K), jnp.int32)]
        + [pltpu.VMEM((_K, d), jnp.float32)] * _NB
        + [pltpu.VMEM((_K,), jnp.int32)] * _NB
        + [pltpu.VMEM_SHARED((n_pad, d), jnp.float32)]
        + [pltpu.SemaphoreType.DMA] * (2 * _NB)
    )

    @functools.partial(
        pl.kernel,
        out_type=jax.ShapeDtypeStruct((_NC, n_pad, d), jnp.float32),
        mesh=mesh,
        scratch_types=scratch,
    )
    def k(hls_hbm, src_hbm, dst_hbm, zrows_hbm, out_hbm, src_all, *rest):
        bufs = rest[:_NB]
        dvecs = rest[_NB:2 * _NB]
        agg_sh = rest[2 * _NB]
        sem_g = rest[2 * _NB + 1:3 * _NB + 1]
        sem_i = rest[3 * _NB + 1:4 * _NB + 1]
        c = lax.axis_index("c")
        s = lax.axis_index("s")
        w = c * _NS + s
        row0 = s * rows_per_tile
        pltpu.sync_copy(src_hbm.at[pl.ds(w * iters, iters), :], src_all)
        zb = bufs[0]
        pltpu.sync_copy(zrows_hbm, zb)

        def zbody(t, carry):
            pltpu.sync_copy(zb, agg_sh.at[pl.ds(row0 + t * _K, _K), :])
            return carry
        lax.fori_loop(0, rows_per_tile // _K, zbody, 0)
        zrem = rows_per_tile % _K
        if zrem:
            pltpu.sync_copy(
                zb.at[pl.ds(0, zrem), :],
                agg_sh.at[pl.ds(row0 + rows_per_tile - zrem, zrem), :])
        for b_ in range(_NB):
            pltpu.async_copy(hls_hbm.at[src_all.at[b_]], bufs[b_], sem_g[b_])
            pltpu.async_copy(dst_hbm.at[w * iters + b_], dvecs[b_], sem_i[b_])
        plsc.subcore_barrier()

        def body(j, carry):
            for b_ in range(_NB):
                i_ = j * _NB + b_
                pltpu.make_async_copy(
                    hls_hbm.at[src_all.at[i_]], bufs[b_], sem_g[b_]).wait()
                pltpu.make_async_copy(
                    dst_hbm.at[w * iters + i_], dvecs[b_], sem_i[b_]).wait()
                pltpu.sync_copy(bufs[b_], agg_sh.at[dvecs[b_]], add=True)

                @pl.when(j + 1 < nblocks)
                def _():
                    pltpu.async_copy(
                        hls_hbm.at[src_all.at[i_ + _NB]], bufs[b_], sem_g[b_])
                    pltpu.async_copy(
                        dst_hbm.at[w * iters + i_ + _NB], dvecs[b_], sem_i[b_])
            return carry
        lax.fori_loop(0, nblocks, body, 0)
        plsc.subcore_barrier()
        pltpu.sync_copy(
            agg_sh.at[pl.ds(row0, rows_per_tile), :],
            out_hbm.at[c, pl.ds(row0, rows_per_tile), :],
        )

    return k


def _rsqrt(x):
    y = lax.rsqrt(x)
    y = y * (1.5 - 0.5 * x * y * y)
    y = y * (1.5 - 0.5 * x * y * y)
    return y


def _kdinv_body(p0_ref, p1_ref, dinv_ref):
    deg = p0_ref[:, 0:1] + p1_ref[:, 0:1] + 2.0
    dinv_ref[...] = _rsqrt(deg)


def _k1_body(h_ref, w_ref, pw_ref, dinv_ref, hls_ref, hl2_ref, r_ref):
    h = h_ref[...]
    hl = jnp.dot(h, w_ref[...], preferred_element_type=jnp.float32,
                 precision=lax.Precision.DEFAULT)
    r = jnp.dot(h, pw_ref[...], preferred_element_type=jnp.float32,
                precision=lax.Precision.DEFAULT)
    dinv = dinv_ref[...]
    hls_ref[...] = hl * dinv
    hl2_ref[...] = hl * (2.0 * dinv * dinv)
    r_ref[...] = r


def _k2a_body(n_real, blk, p0_ref, p1_ref, hl2_ref, b_ref, dinv_ref,
              agg_ref, stats_ref):
    i = pl.program_id(0)
    agg = dinv_ref[...] * (p0_ref[...] + p1_ref[...]) + hl2_ref[...] + b_ref[...]
    agg_ref[...] = agg
    rows = lax.broadcasted_iota(jnp.int32, (blk, 1), 0) + i * blk
    mask = (rows < n_real).astype(jnp.float32)
    am = agg * mask

    @pl.when(i == 0)
    def _():
        stats_ref[...] = jnp.zeros_like(stats_ref)

    stats_ref[0:1, :] = stats_ref[0:1, :] + jnp.sum(am, axis=0, keepdims=True)
    stats_ref[1:2, :] = stats_ref[1:2, :] + jnp.sum(agg * am, axis=0, keepdims=True)


def _k2b_body(n_real, agg_ref, stats_ref, r_ref, g_ref, be_ref, pb_ref,
              lng_ref, lnb_ref, out_ref):
    inv_n = 1.0 / n_real
    mu = stats_ref[0:1, :] * inv_n
    var = stats_ref[1:2, :] * inv_n - mu * mu
    hb = (agg_ref[...] - mu) * _rsqrt(var + 1e-5) * g_ref[...] + be_ref[...]
    hb = hb + r_ref[...] + pb_ref[...]
    hr = jnp.maximum(hb, 0.0)
    m2 = jnp.mean(hr, axis=1, keepdims=True)
    v2 = jnp.mean(hr * hr, axis=1, keepdims=True) - m2 * m2
    out_ref[...] = (hr - m2) * _rsqrt(v2 + 1e-5) * lng_ref[...] + lnb_ref[...]


def kernel(x, edge_index, W, b, bn_gamma, bn_beta, Pw, Pb, ln_g, ln_b):
    n, d = x.shape
    num_layers = W.shape[0]
    e = edge_index.shape[1]

    n_pad = _ceil_to(n + 1, 128)
    chunk_mult = _NB * 8 // _gcd(_NB, 8)
    e_pad = _ceil_to(e, _NC * _NS * _K * chunk_mult)
    grid_n = 8
    blk = n_pad // grid_n

    x_p = jnp.pad(x, ((0, n_pad - n), (0, 0)))
    if e_pad > e:
        pad = jnp.full((2, e_pad - e), n, dtype=edge_index.dtype)
        ei = jnp.concatenate([edge_index, pad], axis=1)
    else:
        ei = edge_index
    src = ei[0].reshape(e_pad // _K, _K)
    dst = ei[1].reshape(e_pad // _K, _K)

    ones_blk = jnp.ones((_K, d), jnp.float32)
    zrows = jnp.zeros((_K, d), jnp.float32)
    deg_p = _sc_degree(n_pad, e_pad, d)(ones_blk, dst, zrows)

    row_spec = pl.BlockSpec((blk, d), lambda i: (i, 0))
    col1_spec = pl.BlockSpec((blk, 1), lambda i: (i, 0))
    deg_spec = row_spec
    full_spec = pl.BlockSpec((d, d), lambda i: (0, 0))
    vec_spec = pl.BlockSpec((1, d), lambda i: (0, 0))
    stats_spec = pl.BlockSpec((8, d), lambda i: (0, 0))

    dinv = pl.pallas_call(
        _kdinv_body,
        grid=(grid_n,),
        in_specs=[deg_spec, deg_spec],
        out_specs=col1_spec,
        out_shape=jax.ShapeDtypeStruct((n_pad, 1), jnp.float32),
    )(deg_p[0], deg_p[1])

    sc_agg = _sc_aggregate(n_pad, e_pad, d)

    h = x_p
    for i in range(num_layers):
        hls, hl2, r = pl.pallas_call(
            _k1_body,
            grid=(grid_n,),
            in_specs=[row_spec, full_spec, full_spec, col1_spec],
            out_specs=[row_spec, row_spec, row_spec],
            out_shape=[jax.ShapeDtypeStruct((n_pad, d), jnp.float32)] * 3,
        )(h, W[i], Pw[i], dinv)

        agg_p = sc_agg(hls, src, dst, zrows)

        agg, stats = pl.pallas_call(
            functools.partial(_k2a_body, n, blk),
            grid=(grid_n,),
            in_specs=[row_spec, row_spec, row_spec, vec_spec, col1_spec],
            out_specs=[row_spec, stats_spec],
            out_shape=[
                jax.ShapeDtypeStruct((n_pad, d), jnp.float32),
                jax.ShapeDtypeStruct((8, d), jnp.float32),
            ],
        )(agg_p[0], agg_p[1], hl2, b[i][None, :], dinv)

        h = pl.pallas_call(
            functools.partial(_k2b_body, float(n)),
            grid=(grid_n,),
            in_specs=[row_spec, stats_spec, row_spec, vec_spec, vec_spec,
                      vec_spec, vec_spec, vec_spec],
            out_specs=row_spec,
            out_shape=jax.ShapeDtypeStruct((n_pad, d), jnp.float32),
        )(agg, stats, r, bn_gamma[i][None, :], bn_beta[i][None, :],
          Pb[i][None, :], ln_g[None, :], ln_b[None, :])

    return h[:n]


# speedup vs baseline: 2.6947x; 2.6947x over previous
import functools

import jax
import jax.numpy as jnp
from jax import lax
from jax.experimental import pallas as pl
from jax.experimental.pallas import tpu as pltpu
from jax.experimental.pallas import tpu_sc as plsc

_NC = 2
_NS = 16
_K = 128
_NB = 2


def _ceil_to(a, m):
    return (a + m - 1) // m * m


def _gcd(a, b):
    while b:
        a, b = b, a % b
    return a


def _sc_degree(n_pad, e_pad, d):
    nw = _NC * _NS
    per_w = e_pad // nw
    iters = per_w // _K
    nblocks = iters // _NB
    rows_per_tile = n_pad // _NS
    mesh = plsc.VectorSubcoreMesh(core_axis_name="c", subcore_axis_name="s")

    scratch = (
        [pltpu.VMEM((_K, d), jnp.float32)] * 2
        + [pltpu.VMEM((_K,), jnp.int32)] * _NB
        + [pltpu.VMEM_SHARED((n_pad, d), jnp.float32)]
        + [pltpu.SemaphoreType.DMA] * _NB
    )

    @functools.partial(
        pl.kernel,
        out_type=jax.ShapeDtypeStruct((_NC, n_pad, d), jnp.float32),
        mesh=mesh,
        scratch_types=scratch,
    )
    def k(ones_hbm, dst_hbm, zrows_hbm, out_hbm, ones_v, zb, *rest):
        dvecs = rest[:_NB]
        deg_sh = rest[_NB]
        sem_i = rest[_NB + 1:2 * _NB + 1]
        c = lax.axis_index("c")
        s = lax.axis_index("s")
        w = c * _NS + s
        row0 = s * rows_per_tile
        pltpu.sync_copy(ones_hbm, ones_v)
        pltpu.sync_copy(zrows_hbm, zb)
        for t in range(rows_per_tile // _K):
            pltpu.sync_copy(zb, deg_sh.at[pl.ds(row0 + t * _K, _K), :])
        zrem = rows_per_tile % _K
        if zrem:
            pltpu.sync_copy(
                zb.at[pl.ds(0, zrem), :],
                deg_sh.at[pl.ds(row0 + rows_per_tile - zrem, zrem), :])
        for b_ in range(_NB):
            pltpu.async_copy(dst_hbm.at[w * iters + b_], dvecs[b_], sem_i[b_])
        plsc.subcore_barrier()

        def body(j, carry):
            for b_ in range(_NB):
                i_ = j * _NB + b_
                pltpu.make_async_copy(
                    dst_hbm.at[w * iters + i_], dvecs[b_], sem_i[b_]).wait()
                pltpu.sync_copy(ones_v, deg_sh.at[dvecs[b_]], add=True)

                @pl.when(j + 1 < nblocks)
                def _():
                    pltpu.async_copy(
                        dst_hbm.at[w * iters + i_ + _NB], dvecs[b_], sem_i[b_])
            return carry
        lax.fori_loop(0, nblocks, body, 0)
        plsc.subcore_barrier()
        pltpu.sync_copy(
            deg_sh.at[pl.ds(row0, rows_per_tile), :],
            out_hbm.at[c, pl.ds(row0, rows_per_tile), :],
        )

    return k


def _sc_aggregate(n_pad, e_pad, d):
    nw = _NC * _NS
    per_w = e_pad // nw
    iters = per_w // _K
    nblocks = iters // _NB
    rows_per_tile = n_pad // _NS
    mesh = plsc.VectorSubcoreMesh(core_axis_name="c", subcore_axis_name="s")

    scratch = (
        [pltpu.VMEM((iters, _K), jnp.int32)]
        + [pltpu.VMEM((_K, d), jnp.float32)] * _NB
        + [pltpu.VMEM((_K,), jnp.int32)] * _NB
        + [pltpu.VMEM_SHARED((n_pad, d), jnp.float32)]
        + [pltpu.SemaphoreType.DMA] * (2 * _NB)
    )

    @functools.partial(
        pl.kernel,
        out_type=jax.ShapeDtypeStruct((_NC, n_pad, d), jnp.float32),
        mesh=mesh,
        scratch_types=scratch,
    )
    def k(hls_hbm, src_hbm, dst_hbm, zrows_hbm, out_hbm, src_all, *rest):
        bufs = rest[:_NB]
        dvecs = rest[_NB:2 * _NB]
        agg_sh = rest[2 * _NB]
        sem_g = rest[2 * _NB + 1:3 * _NB + 1]
        sem_i = rest[3 * _NB + 1:4 * _NB + 1]
        c = lax.axis_index("c")
        s = lax.axis_index("s")
        w = c * _NS + s
        row0 = s * rows_per_tile
        pltpu.sync_copy(src_hbm.at[pl.ds(w * iters, iters), :], src_all)
        zb = bufs[0]
        pltpu.sync_copy(zrows_hbm, zb)

        def zbody(t, carry):
            pltpu.sync_copy(zb, agg_sh.at[pl.ds(row0 + t * _K, _K), :])
            return carry
        lax.fori_loop(0, rows_per_tile // _K, zbody, 0)
        zrem = rows_per_tile % _K
        if zrem:
            pltpu.sync_copy(
                zb.at[pl.ds(0, zrem), :],
                agg_sh.at[pl.ds(row0 + rows_per_tile - zrem, zrem), :])
        for b_ in range(_NB):
            pltpu.async_copy(hls_hbm.at[src_all.at[b_]], bufs[b_], sem_g[b_])
            pltpu.async_copy(dst_hbm.at[w * iters + b_], dvecs[b_], sem_i[b_])
        plsc.subcore_barrier()

        def body(j, carry):
            for b_ in range(_NB):
                i_ = j * _NB + b_
                pltpu.make_async_copy(
                    hls_hbm.at[src_all.at[i_]], bufs[b_], sem_g[b_]).wait()
                pltpu.make_async_copy(
                    dst_hbm.at[w * iters + i_], dvecs[b_], sem_i[b_]).wait()
                pltpu.sync_copy(bufs[b_], agg_sh.at[dvecs[b_]], add=True)

                @pl.when(j + 1 < nblocks)
                def _():
                    pltpu.async_copy(
                        hls_hbm.at[src_all.at[i_ + _NB]], bufs[b_], sem_g[b_])
                    pltpu.async_copy(
                        dst_hbm.at[w * iters + i_ + _NB], dvecs[b_], sem_i[b_])
            return carry
        lax.fori_loop(0, nblocks, body, 0)
        plsc.subcore_barrier()
        pltpu.sync_copy(
            agg_sh.at[pl.ds(row0, rows_per_tile), :],
            out_hbm.at[c, pl.ds(row0, rows_per_tile), :],
        )

    return k


def _rsqrt(x):
    y = lax.rsqrt(x)
    y = y * (1.5 - 0.5 * x * y * y)
    y = y * (1.5 - 0.5 * x * y * y)
    return y


def _kdinv_body(p0_ref, p1_ref, dinv_ref):
    deg = p0_ref[:, 0:1] + p1_ref[:, 0:1] + 2.0
    dinv_ref[...] = _rsqrt(deg)


def _k1_body(h_ref, w_ref, pw_ref, dinv_ref, hls_ref, hl2_ref, r_ref):
    h = h_ref[...]
    hl = jnp.dot(h, w_ref[...], preferred_element_type=jnp.float32,
                 precision=lax.Precision.DEFAULT)
    r = jnp.dot(h, pw_ref[...], preferred_element_type=jnp.float32,
                precision=lax.Precision.DEFAULT)
    dinv = dinv_ref[...]
    hls_ref[...] = hl * dinv
    hl2_ref[...] = hl * (2.0 * dinv * dinv)
    r_ref[...] = r


def _k2a_body(n_real, blk, p0_ref, p1_ref, hl2_ref, b_ref, dinv_ref,
              agg_ref, stats_ref):
    i = pl.program_id(0)
    agg = dinv_ref[...] * (p0_ref[...] + p1_ref[...]) + hl2_ref[...] + b_ref[...]
    agg_ref[...] = agg
    rows = lax.broadcasted_iota(jnp.int32, (blk, 1), 0) + i * blk
    mask = (rows < n_real).astype(jnp.float32)
    am = agg * mask

    @pl.when(i == 0)
    def _():
        stats_ref[...] = jnp.zeros_like(stats_ref)

    stats_ref[0:1, :] = stats_ref[0:1, :] + jnp.sum(am, axis=0, keepdims=True)
    stats_ref[1:2, :] = stats_ref[1:2, :] + jnp.sum(agg * am, axis=0, keepdims=True)


def _k2b_body(n_real, agg_ref, stats_ref, r_ref, g_ref, be_ref, pb_ref,
              lng_ref, lnb_ref, out_ref):
    inv_n = 1.0 / n_real
    mu = stats_ref[0:1, :] * inv_n
    var = stats_ref[1:2, :] * inv_n - mu * mu
    hb = (agg_ref[...] - mu) * _rsqrt(var + 1e-5) * g_ref[...] + be_ref[...]
    hb = hb + r_ref[...] + pb_ref[...]
    hr = jnp.maximum(hb, 0.0)
    m2 = jnp.mean(hr, axis=1, keepdims=True)
    v2 = jnp.mean(hr * hr, axis=1, keepdims=True) - m2 * m2
    out_ref[...] = (hr - m2) * _rsqrt(v2 + 1e-5) * lng_ref[...] + lnb_ref[...]


def kernel(x, edge_index, W, b, bn_gamma, bn_beta, Pw, Pb, ln_g, ln_b):
    n, d = x.shape
    num_layers = W.shape[0]
    e = edge_index.shape[1]

    n_pad = _ceil_to(n + 1, 128)
    chunk_mult = _NB * 8 // _gcd(_NB, 8)
    e_pad = _ceil_to(e, _NC * _NS * _K * chunk_mult)
    grid_n = 8
    blk = n_pad // grid_n

    x_p = jnp.pad(x, ((0, n_pad - n), (0, 0)))
    if e_pad > e:
        pad = jnp.full((2, e_pad - e), n, dtype=edge_index.dtype)
        ei = jnp.concatenate([edge_index, pad], axis=1)
    else:
        ei = edge_index
    src = ei[0].reshape(e_pad // _K, _K)
    dst = ei[1].reshape(e_pad // _K, _K)

    ones_blk = jnp.ones((_K, d), jnp.float32)
    zrows = jnp.zeros((_K, d), jnp.float32)
    deg_p = _sc_degree(n_pad, e_pad, d)(ones_blk, dst, zrows)

    row_spec = pl.BlockSpec((blk, d), lambda i: (i, 0))
    col1_spec = pl.BlockSpec((blk, 1), lambda i: (i, 0))
    deg_spec = row_spec
    full_spec = pl.BlockSpec((d, d), lambda i: (0, 0))
    vec_spec = pl.BlockSpec((1, d), lambda i: (0, 0))
    stats_spec = pl.BlockSpec((8, d), lambda i: (0, 0))

    dinv = pl.pallas_call(
        _kdinv_body,
        grid=(grid_n,),
        in_specs=[deg_spec, deg_spec],
        out_specs=col1_spec,
        out_shape=jax.ShapeDtypeStruct((n_pad, 1), jnp.float32),
    )(deg_p[0], deg_p[1])

    sc_agg = _sc_aggregate(n_pad, e_pad, d)

    h = x_p
    for i in range(num_layers):
        hls, hl2, r = pl.pallas_call(
            _k1_body,
            grid=(grid_n,),
            in_specs=[row_spec, full_spec, full_spec, col1_spec],
            out_specs=[row_spec, row_spec, row_spec],
            out_shape=[jax.ShapeDtypeStruct((n_pad, d), jnp.float32)] * 3,
        )(h, W[i], Pw[i], dinv)

        agg_p = sc_agg(hls, src, dst, zrows)

        agg, stats = pl.pallas_call(
            functools.partial(_k2a_body, n, blk),
            grid=(grid_n,),
            in_specs=[row_spec, row_spec, row_spec, vec_spec, col1_spec],
            out_specs=[row_spec, stats_spec],
            out_shape=[
                jax.ShapeDtypeStruct((n_pad, d), jnp.float32),
                jax.ShapeDtypeStruct((8, d), jnp.float32),
            ],
        )(agg_p[0], agg_p[1], hl2, b[i][None, :], dinv)

        h = pl.pallas_call(
            functools.partial(_k2b_body, float(n)),
            grid=(grid_n,),
            in_specs=[row_spec, stats_spec, row_spec, vec_spec, vec_spec,
                      vec_spec, vec_spec, vec_spec],
            out_specs=row_spec,
            out_shape=jax.ShapeDtypeStruct((n_pad, d), jnp.float32),
        )(agg, stats, r, bn_gamma[i][None, :], bn_beta[i][None, :],
          Pb[i][None, :], ln_g[None, :], ln_b[None, :])

    return h[:n]


# speedup vs baseline: 8.5262x; 3.1641x over previous
import functools

import jax
import jax.numpy as jnp
from jax import lax
from jax.experimental import pallas as pl
from jax.experimental.pallas import tpu as pltpu
from jax.experimental.pallas import tpu_sc as plsc

_NC = 2
_NS = 16
_K = 128
_NB = 2


def _ceil_to(a, m):
    return (a + m - 1) // m * m


def _gcd(a, b):
    while b:
        a, b = b, a % b
    return a


def _sc_degree(n_pad, e_pad, d):
    nw = _NC * _NS
    per_w = e_pad // nw
    iters = per_w // _K
    nblocks = iters // _NB
    rows_per_tile = n_pad // _NS
    mesh = plsc.VectorSubcoreMesh(core_axis_name="c", subcore_axis_name="s")

    scratch = (
        [pltpu.VMEM((_K, d), jnp.float32)] * 2
        + [pltpu.VMEM((_K,), jnp.int32)] * _NB
        + [pltpu.VMEM_SHARED((n_pad, d), jnp.float32)]
        + [pltpu.SemaphoreType.DMA] * _NB
    )

    @functools.partial(
        pl.kernel,
        out_type=jax.ShapeDtypeStruct((_NC, n_pad, d), jnp.float32),
        mesh=mesh,
        scratch_types=scratch,
    )
    def k(ones_hbm, dst_hbm, zrows_hbm, out_hbm, ones_v, zb, *rest):
        dvecs = rest[:_NB]
        deg_sh = rest[_NB]
        sem_i = rest[_NB + 1:2 * _NB + 1]
        c = lax.axis_index("c")
        s = lax.axis_index("s")
        w = c * _NS + s
        row0 = s * rows_per_tile
        pltpu.sync_copy(ones_hbm, ones_v)
        pltpu.sync_copy(zrows_hbm, zb)
        for t in range(rows_per_tile // _K):
            pltpu.sync_copy(zb, deg_sh.at[pl.ds(row0 + t * _K, _K), :])
        zrem = rows_per_tile % _K
        if zrem:
            pltpu.sync_copy(
                zb.at[pl.ds(0, zrem), :],
                deg_sh.at[pl.ds(row0 + rows_per_tile - zrem, zrem), :])
        for b_ in range(_NB):
            pltpu.async_copy(dst_hbm.at[w * iters + b_], dvecs[b_], sem_i[b_])
        plsc.subcore_barrier()

        def body(j, carry):
            for b_ in range(_NB):
                i_ = j * _NB + b_
                pltpu.make_async_copy(
                    dst_hbm.at[w * iters + i_], dvecs[b_], sem_i[b_]).wait()
                pltpu.sync_copy(ones_v, deg_sh.at[dvecs[b_]], add=True)

                @pl.when(j + 1 < nblocks)
                def _():
                    pltpu.async_copy(
                        dst_hbm.at[w * iters + i_ + _NB], dvecs[b_], sem_i[b_])
            return carry
        lax.fori_loop(0, nblocks, body, 0)
        plsc.subcore_barrier()
        pltpu.sync_copy(
            deg_sh.at[pl.ds(row0, rows_per_tile), :],
            out_hbm.at[c, pl.ds(row0, rows_per_tile), :],
        )

    return k


def _sc_aggregate(n_pad, e_pad, d):
    nw = _NC * _NS
    per_w = e_pad // nw
    iters = per_w // _K
    nblocks = iters // _NB
    rows_per_tile = n_pad // _NS
    mesh = plsc.VectorSubcoreMesh(core_axis_name="c", subcore_axis_name="s")

    scratch = (
        [pltpu.VMEM((iters, _K), jnp.int32)]
        + [pltpu.VMEM((_K, d), jnp.float32)] * _NB
        + [pltpu.VMEM((_K,), jnp.int32)] * _NB
        + [pltpu.VMEM_SHARED((n_pad, d), jnp.float32)]
        + [pltpu.SemaphoreType.DMA] * (2 * _NB)
    )

    @functools.partial(
        pl.kernel,
        out_type=jax.ShapeDtypeStruct((_NC, n_pad, d), jnp.float32),
        mesh=mesh,
        scratch_types=scratch,
    )
    def k(hls_hbm, src_hbm, dst_hbm, zrows_hbm, out_hbm, src_all, *rest):
        bufs = rest[:_NB]
        dvecs = rest[_NB:2 * _NB]
        agg_sh = rest[2 * _NB]
        sem_g = rest[2 * _NB + 1:3 * _NB + 1]
        sem_i = rest[3 * _NB + 1:4 * _NB + 1]
        c = lax.axis_index("c")
        s = lax.axis_index("s")
        w = c * _NS + s
        row0 = s * rows_per_tile
        pltpu.sync_copy(src_hbm.at[pl.ds(w * iters, iters), :], src_all)
        zb = bufs[0]
        pltpu.sync_copy(zrows_hbm, zb)

        def zbody(t, carry):
            pltpu.sync_copy(zb, agg_sh.at[pl.ds(row0 + t * _K, _K), :])
            return carry
        lax.fori_loop(0, rows_per_tile // _K, zbody, 0)
        zrem = rows_per_tile % _K
        if zrem:
            pltpu.sync_copy(
                zb.at[pl.ds(0, zrem), :],
                agg_sh.at[pl.ds(row0 + rows_per_tile - zrem, zrem), :])
        for b_ in range(_NB):
            pltpu.async_copy(hls_hbm.at[src_all.at[b_]], bufs[b_], sem_g[b_])
            pltpu.async_copy(dst_hbm.at[w * iters + b_], dvecs[b_], sem_i[b_])
        plsc.subcore_barrier()

        def body(j, carry):
            for b_ in range(_NB):
                i_ = j * _NB + b_
                pltpu.make_async_copy(
                    hls_hbm.at[src_all.at[i_]], bufs[b_], sem_g[b_]).wait()
                pltpu.make_async_copy(
                    dst_hbm.at[w * iters + i_], dvecs[b_], sem_i[b_]).wait()
                pltpu.sync_copy(bufs[b_], agg_sh.at[dvecs[b_]], add=True)

                @pl.when(j + 1 < nblocks)
                def _():
                    pltpu.async_copy(
                        hls_hbm.at[src_all.at[i_ + _NB]], bufs[b_], sem_g[b_])
                    pltpu.async_copy(
                        dst_hbm.at[w * iters + i_ + _NB], dvecs[b_], sem_i[b_])
            return carry
        lax.fori_loop(0, nblocks, body, 0)
        plsc.subcore_barrier()
        pltpu.sync_copy(
            agg_sh.at[pl.ds(row0, rows_per_tile), :],
            out_hbm.at[c, pl.ds(row0, rows_per_tile), :],
        )

    return k


def _rsqrt(x):
    y = lax.rsqrt(x)
    y = y * (1.5 - 0.5 * x * y * y)
    y = y * (1.5 - 0.5 * x * y * y)
    return y


def _kdinv_body(p0_ref, p1_ref, dinv_ref):
    deg = p0_ref[:, 0:1] + p1_ref[:, 0:1] + 2.0
    dinv_ref[...] = _rsqrt(deg)


def _k1_body(h_ref, w_ref, pw_ref, dinv_ref, hls_ref, hl2_ref, r_ref):
    h = h_ref[...]
    hl = jnp.dot(h, w_ref[...], preferred_element_type=jnp.float32,
                 precision=lax.Precision.DEFAULT)
    r = jnp.dot(h, pw_ref[...], preferred_element_type=jnp.float32,
                precision=lax.Precision.DEFAULT)
    dinv = dinv_ref[...]
    hls_ref[...] = hl * dinv
    hl2_ref[...] = hl * (2.0 * dinv * dinv)
    r_ref[...] = r


def _k2a_body(n_real, blk, p0_ref, p1_ref, hl2_ref, b_ref, dinv_ref,
              agg_ref, stats_ref):
    i = pl.program_id(0)
    agg = dinv_ref[...] * (p0_ref[...] + p1_ref[...]) + hl2_ref[...] + b_ref[...]
    agg_ref[...] = agg
    rows = lax.broadcasted_iota(jnp.int32, (blk, 1), 0) + i * blk
    mask = (rows < n_real).astype(jnp.float32)
    am = agg * mask

    @pl.when(i == 0)
    def _():
        stats_ref[...] = jnp.zeros_like(stats_ref)

    stats_ref[0:1, :] = stats_ref[0:1, :] + jnp.sum(am, axis=0, keepdims=True)
    stats_ref[1:2, :] = stats_ref[1:2, :] + jnp.sum(agg * am, axis=0, keepdims=True)


def _k2b_body(n_real, agg_ref, stats_ref, r_ref, g_ref, be_ref, pb_ref,
              lng_ref, lnb_ref, out_ref):
    inv_n = 1.0 / n_real
    mu = stats_ref[0:1, :] * inv_n
    var = stats_ref[1:2, :] * inv_n - mu * mu
    hb = (agg_ref[...] - mu) * _rsqrt(var + 1e-5) * g_ref[...] + be_ref[...]
    hb = hb + r_ref[...] + pb_ref[...]
    hr = jnp.maximum(hb, 0.0)
    m2 = jnp.mean(hr, axis=1, keepdims=True)
    v2 = jnp.mean(hr * hr, axis=1, keepdims=True) - m2 * m2
    out_ref[...] = (hr - m2) * _rsqrt(v2 + 1e-5) * lng_ref[...] + lnb_ref[...]


def kernel(x, edge_index, W, b, bn_gamma, bn_beta, Pw, Pb, ln_g, ln_b):
    n, d = x.shape
    num_layers = W.shape[0]
    e = edge_index.shape[1]

    n_pad = _ceil_to(n + 1, 128)
    chunk_mult = _NB * 8 // _gcd(_NB, 8)
    e_pad = _ceil_to(e, _NC * _NS * _K * chunk_mult)
    grid_n = 8
    blk = n_pad // grid_n

    x_p = jnp.pad(x, ((0, n_pad - n), (0, 0)))
    if e_pad > e:
        pidx = n + (jnp.arange(e_pad - e, dtype=edge_index.dtype)
                    % (n_pad - n))
        ei = jnp.concatenate([edge_index, jnp.stack([pidx, pidx])], axis=1)
    else:
        ei = edge_index
    src = ei[0].reshape(e_pad // _K, _K)
    dst = ei[1].reshape(e_pad // _K, _K)

    ones_blk = jnp.ones((_K, d), jnp.float32)
    zrows = jnp.zeros((_K, d), jnp.float32)
    deg_p = _sc_degree(n_pad, e_pad, d)(ones_blk, dst, zrows)

    row_spec = pl.BlockSpec((blk, d), lambda i: (i, 0))
    col1_spec = pl.BlockSpec((blk, 1), lambda i: (i, 0))
    deg_spec = row_spec
    full_spec = pl.BlockSpec((d, d), lambda i: (0, 0))
    vec_spec = pl.BlockSpec((1, d), lambda i: (0, 0))
    stats_spec = pl.BlockSpec((8, d), lambda i: (0, 0))

    dinv = pl.pallas_call(
        _kdinv_body,
        grid=(grid_n,),
        in_specs=[deg_spec, deg_spec],
        out_specs=col1_spec,
        out_shape=jax.ShapeDtypeStruct((n_pad, 1), jnp.float32),
    )(deg_p[0], deg_p[1])

    sc_agg = _sc_aggregate(n_pad, e_pad, d)

    h = x_p
    for i in range(num_layers):
        hls, hl2, r = pl.pallas_call(
            _k1_body,
            grid=(grid_n,),
            in_specs=[row_spec, full_spec, full_spec, col1_spec],
            out_specs=[row_spec, row_spec, row_spec],
            out_shape=[jax.ShapeDtypeStruct((n_pad, d), jnp.float32)] * 3,
        )(h, W[i], Pw[i], dinv)

        agg_p = sc_agg(hls, src, dst, zrows)

        agg, stats = pl.pallas_call(
            functools.partial(_k2a_body, n, blk),
            grid=(grid_n,),
            in_specs=[row_spec, row_spec, row_spec, vec_spec, col1_spec],
            out_specs=[row_spec, stats_spec],
            out_shape=[
                jax.ShapeDtypeStruct((n_pad, d), jnp.float32),
                jax.ShapeDtypeStruct((8, d), jnp.float32),
            ],
        )(agg_p[0], agg_p[1], hl2, b[i][None, :], dinv)

        h = pl.pallas_call(
            functools.partial(_k2b_body, float(n)),
            grid=(grid_n,),
            in_specs=[row_spec, stats_spec, row_spec, vec_spec, vec_spec,
                      vec_spec, vec_spec, vec_spec],
            out_specs=row_spec,
            out_shape=jax.ShapeDtypeStruct((n_pad, d), jnp.float32),
        )(agg, stats, r, bn_gamma[i][None, :], bn_beta[i][None, :],
          Pb[i][None, :], ln_g[None, :], ln_b[None, :])

    return h[:n]
